# Initial kernel scaffold; baseline (speedup 1.0000x reference)
import jax
import jax.numpy as jnp
from jax.experimental import pallas as pl


def kernel(h, m_ij, x, vel_all, edge_index, W1, b1, W2, b2, G1, gb1, G2, gb2):
    raise NotImplementedError("write your pallas kernel here")


# speedup vs baseline: 9.2184x; 9.2184x over previous
import functools

import numpy as np
import jax
import jax.numpy as jnp
from jax import lax
from jax.experimental import pallas as pl
from jax.experimental.pallas import tpu as pltpu
from jax.experimental.pallas import tpu_sc as plsc

_N = 10000
_E = 320000
_D = 128

_NW = 32
_EPW = 10240
_EPAD = _NW * _EPW
_G = 128
_GPW = _EPW // _G
_CH = 1024
_GPC = _CH // _G
_NCH = _EPW // _CH
_NPAD = 10240

_ROWS_PER_SUB = _NPAD // 16


def _edge_mlp_body(m_ref, w1t_ref, b1_ref, w2c_ref, b2_ref, o_ref):
    a = jnp.dot(m_ref[...], w1t_ref[...], preferred_element_type=jnp.float32)
    a = a + b1_ref[...]
    a = a * lax.logistic(a)
    o_ref[...] = (
        jnp.dot(a, w2c_ref[...], preferred_element_type=jnp.float32) + b2_ref[...]
    )


def _edge_w(m_ij, W1, b1, W2, b2):
    BE = 2560
    return pl.pallas_call(
        _edge_mlp_body,
        grid=(_E // BE,),
        in_specs=[
            pl.BlockSpec((BE, _D), lambda i: (i, 0)),
            pl.BlockSpec((_D, _D), lambda i: (0, 0)),
            pl.BlockSpec((1, _D), lambda i: (0, 0)),
            pl.BlockSpec((_D, 1), lambda i: (0, 0)),
            pl.BlockSpec((1, 1), lambda i: (0, 0)),
        ],
        out_specs=pl.BlockSpec((BE, 1), lambda i: (i, 0)),
        out_shape=jax.ShapeDtypeStruct((_E, 1), jnp.float32),
    )(m_ij, W1.T, b1.reshape(1, _D), W2.T, b2.reshape(1, 1))


def _sc_scatter(xflat, src_pad, w_pad, dst3, zeros8):
    mesh = plsc.VectorSubcoreMesh(core_axis_name="c", subcore_axis_name="s")

    @functools.partial(
        pl.kernel,
        mesh=mesh,
        compiler_params=pltpu.CompilerParams(
            needs_layout_passes=False, use_tc_tiling_on_sc=False),
        out_type=jax.ShapeDtypeStruct((2, _NPAD, 8), jnp.float32),
        scratch_types=[
            pltpu.VMEM((3 * _N,), jnp.float32),
            pltpu.VMEM((_EPW,), jnp.int32),
            pltpu.VMEM((_EPW,), jnp.float32),
            pltpu.VMEM((_GPW, _G), jnp.int32),
            pltpu.VMEM((_CH, 8), jnp.float32),
            pltpu.VMEM_SHARED((_NPAD, 8), jnp.float32),
        ],
    )
    def k(x_hbm, src_hbm, w_hbm, dst_hbm, z_hbm, out_hbm,
          x_v, src_v, w_v, dst_v, vals_v, acc_sh):
        c = lax.axis_index("c")
        s = lax.axis_index("s")
        wid = s * 2 + c
        base = wid * _EPW

        pltpu.sync_copy(z_hbm.at[pl.ds(0, _ROWS_PER_SUB)],
                        acc_sh.at[pl.ds(s * _ROWS_PER_SUB, _ROWS_PER_SUB)])
        pltpu.sync_copy(z_hbm, vals_v)
        pltpu.sync_copy(x_hbm, x_v)
        pltpu.sync_copy(src_hbm.at[pl.ds(base, _EPW)], src_v)
        pltpu.sync_copy(w_hbm.at[pl.ds(base, _EPW)], w_v)
        pltpu.sync_copy(dst_hbm.at[wid], dst_v)
        plsc.subcore_barrier()

        iota16 = lax.iota(jnp.int32, 16)
        onesf = jnp.ones((16,), jnp.float32)
        col0 = jnp.full((16,), 0, jnp.int32)
        col1 = jnp.full((16,), 1, jnp.int32)
        col2 = jnp.full((16,), 2, jnp.int32)
        col3 = jnp.full((16,), 3, jnp.int32)
        col4 = jnp.full((16,), 4, jnp.int32)

        def chunk_body(ci, carry):
            off = ci * _CH

            def grp(g, carry2):
                e0 = off + g * 16
                src16 = src_v[pl.ds(e0, 16)]
                w16 = w_v[pl.ds(e0, 16)]
                row = g * 16 + iota16
                xs0 = plsc.load_gather(x_v, [src16])
                xs1 = plsc.load_gather(x_v, [src16 + _N])
                xs2 = plsc.load_gather(x_v, [src16 + 2 * _N])
                plsc.store_scatter(vals_v, [row, col0], xs0 * w16)
                plsc.store_scatter(vals_v, [row, col1], xs1 * w16)
                plsc.store_scatter(vals_v, [row, col2], xs2 * w16)
                plsc.store_scatter(vals_v, [row, col3], w16)
                plsc.store_scatter(vals_v, [row, col4], onesf)
                return carry2

            lax.fori_loop(0, _CH // 16, grp, 0)

            def scat(g2, carry3):
                pltpu.sync_copy(vals_v.at[pl.ds(g2 * _G, _G)],
                                acc_sh.at[dst_v.at[ci * _GPC + g2]],
                                add=True)
                return carry3

            lax.fori_loop(0, _GPC, scat, 0)
            return carry

        lax.fori_loop(0, _NCH, chunk_body, 0)
        plsc.subcore_barrier()
        pltpu.sync_copy(acc_sh.at[pl.ds(s * _ROWS_PER_SUB, _ROWS_PER_SUB)],
                        out_hbm.at[c, pl.ds(s * _ROWS_PER_SUB, _ROWS_PER_SUB)])

    return k(xflat, src_pad, w_pad, dst3, zeros8)


def _final_body(h_ref, g1t_ref, gb1_ref, g2r_ref, gb2r_ref, v15_ref, t_ref,
                x3_ref, s0_ref, s1_ref, o_ref):
    a = jnp.dot(h_ref[...], g1t_ref[...], preferred_element_type=jnp.float32)
    a = a + gb1_ref[...]
    a = a * lax.logistic(a)
    ar = jnp.dot(a, g2r_ref[...], preferred_element_type=jnp.float32)
    ar = ar + gb2r_ref[...]
    prod = ar * v15_ref[...]
    vc = jnp.dot(prod, t_ref[...], preferred_element_type=jnp.float32)
    S = s0_ref[...] + s1_ref[...]
    cnt = jnp.maximum(S[:, 4:5], 1.0)
    geom = (S[:, 0:3] - x3_ref[...] * S[:, 3:4]) / cnt
    o_ref[...] = vc + geom


_TMAT = np.zeros((15, 3), dtype=np.float32)
for _k in range(5):
    for _j in range(3):
        _TMAT[3 * _k + _j, _j] = 1.0


def _final(h, G1, gb1, G2, gb2, vel15, x, S0, S1):
    BN = 2000
    g2rep = jnp.repeat(G2.T, 3, axis=1)
    gb2rep = jnp.repeat(gb2, 3).reshape(1, 15)
    tmat = jnp.asarray(_TMAT)
    return pl.pallas_call(
        _final_body,
        grid=(_N // BN,),
        in_specs=[
            pl.BlockSpec((BN, _D), lambda i: (i, 0)),
            pl.BlockSpec((_D, _D), lambda i: (0, 0)),
            pl.BlockSpec((1, _D), lambda i: (0, 0)),
            pl.BlockSpec((_D, 15), lambda i: (0, 0)),
            pl.BlockSpec((1, 15), lambda i: (0, 0)),
            pl.BlockSpec((BN, 15), lambda i: (i, 0)),
            pl.BlockSpec((15, 3), lambda i: (0, 0)),
            pl.BlockSpec((BN, 3), lambda i: (i, 0)),
            pl.BlockSpec((BN, 8), lambda i: (i, 0)),
            pl.BlockSpec((BN, 8), lambda i: (i, 0)),
        ],
        out_specs=pl.BlockSpec((BN, 3), lambda i: (i, 0)),
        out_shape=jax.ShapeDtypeStruct((_N, 3), jnp.float32),
    )(h, G1.T, gb1.reshape(1, _D), g2rep, gb2rep, vel15, tmat, x, S0, S1)


def kernel(h, m_ij, x, vel_all, edge_index, W1, b1, W2, b2, G1, gb1, G2, gb2):
    w_e = _edge_w(m_ij, W1, b1, W2, b2)

    src = edge_index[0]
    dst = edge_index[1]
    pad = _EPAD - _E
    src_pad = jnp.concatenate([src, jnp.zeros((pad,), jnp.int32)])
    dst_pad = jnp.concatenate([dst, jnp.full((pad,), _N, jnp.int32)])
    w_pad = jnp.concatenate([w_e[:, 0], jnp.zeros((pad,), jnp.float32)])
    dst3 = dst_pad.reshape(_NW, _GPW, _G)
    xflat = x.T.reshape(-1)
    zeros8 = jnp.zeros((_CH, 8), jnp.float32)

    S_sc = _sc_scatter(xflat, src_pad, w_pad, dst3, zeros8)
    S0 = S_sc[0, :_N]
    S1 = S_sc[1, :_N]

    vel15 = vel_all.reshape(_N, 15)
    return _final(h, G1, gb1, G2, gb2, vel15, x, S0, S1)
